# baseline (speedup 1.0000x reference)
import functools

import jax
import jax.numpy as jnp
from jax import lax
from jax.experimental import pallas as pl
from jax.experimental.pallas import tpu as pltpu
from jax.experimental.pallas import tpu_sc as plsc

CHUNK = 128
NBUF = 4
TBLK = 1024


def _retile(table_t):
    d, n = table_t.shape
    grid = (n + TBLK - 1) // TBLK

    def body(x_ref, o_ref):
        o_ref[:, 0:d] = x_ref[...].T
        o_ref[:, d:128] = jnp.zeros((TBLK, 128 - d), jnp.float32)

    return pl.pallas_call(
        body,
        grid=(grid,),
        in_specs=[pl.BlockSpec((d, TBLK), lambda i: (0, i))],
        out_specs=pl.BlockSpec((TBLK, 128), lambda i: (i, 0)),
        out_shape=jax.ShapeDtypeStruct((n, 128), jnp.float32),
    )(table_t)


@functools.lru_cache(maxsize=None)
def _build(n, d):
    info = plsc.get_sparse_core_info()
    nw = info.num_cores * info.num_subcores
    assert n % nw == 0
    per_w = n // nw
    assert per_w % CHUNK == 0
    n_chunks = per_w // CHUNK
    assert n_chunks % NBUF == 0
    n_rounds = n_chunks // NBUF

    mesh = plsc.VectorSubcoreMesh(core_axis_name="c", subcore_axis_name="s")

    @functools.partial(
        pl.kernel,
        out_type=jax.ShapeDtypeStruct((n, 128), jnp.float32),
        mesh=mesh,
        scratch_types=[
            pltpu.VMEM((n_chunks, CHUNK), jnp.int32),
            [pltpu.VMEM((CHUNK, 128), jnp.float32) for _ in range(NBUF)],
            [pltpu.SemaphoreType.DMA for _ in range(NBUF)],
            [pltpu.SemaphoreType.DMA for _ in range(NBUF)],
        ],
        compiler_params=pltpu.CompilerParams(use_tc_tiling_on_sc=False),
    )
    def gather_kernel(idx_hbm, table_hbm, out_hbm, idx_all, rows, gsems,
                      osems):
        wid = lax.axis_index("s") * info.num_cores + lax.axis_index("c")
        base = wid * per_w

        pltpu.sync_copy(idx_hbm.at[wid], idx_all)

        def stage(k, slot):
            pltpu.make_async_copy(table_hbm.at[idx_all.at[k]], rows[slot],
                                  gsems[slot]).start()

        def wait_gather(slot):
            pltpu.make_async_copy(table_hbm.at[idx_all.at[0]], rows[slot],
                                  gsems[slot]).wait()

        def writeback(j, slot):
            pltpu.make_async_copy(rows[slot],
                                  out_hbm.at[pl.ds(base + j * CHUNK,
                                                    CHUNK)],
                                  osems[slot]).start()

        def wait_writeback(slot):
            pltpu.make_async_copy(rows[slot],
                                  out_hbm.at[pl.ds(base, CHUNK)],
                                  osems[slot]).wait()

        for s in range(NBUF - 1):
            stage(s, s)

        def round_body(r, carry):
            j0 = r * NBUF
            for s in range(NBUF):
                j = j0 + s
                prev_slot = (s - 1) % NBUF
                @pl.when(j >= 1)
                def _():
                    wait_writeback(prev_slot)

                @pl.when(j + NBUF - 1 < n_chunks)
                def _():
                    stage(j + NBUF - 1, prev_slot)

                wait_gather(s)
                writeback(j, s)
            return carry

        lax.fori_loop(0, n_rounds, round_body, 0)
        wait_writeback((n_chunks - 1) % NBUF)

    return gather_kernel


def kernel(words, word_embed_table):
    b, s = words.shape
    _, d = word_embed_table.shape
    n = b * s
    info = plsc.get_sparse_core_info()
    nw = info.num_cores * info.num_subcores
    per_w = n // nw
    idx3 = words.reshape(nw, per_w // CHUNK, CHUNK).astype(jnp.int32)
    n_words = word_embed_table.shape[0]
    table128 = _retile(word_embed_table.T)
    out = _build(n, d)(idx3, table128)
    return out[:, :d].reshape(b, s, d)


# speedup vs baseline: 1.2917x; 1.2917x over previous
import functools

import jax
import jax.numpy as jnp
from jax import lax
from jax.experimental import pallas as pl
from jax.experimental.pallas import tpu as pltpu
from jax.experimental.pallas import tpu_sc as plsc

CHUNK = 128
NBUF = 8
@functools.lru_cache(maxsize=None)
def _build(n, d):
    info = plsc.get_sparse_core_info()
    nw = info.num_cores * info.num_subcores
    assert n % nw == 0
    per_w = n // nw
    assert per_w % CHUNK == 0
    n_chunks = per_w // CHUNK
    assert n_chunks % NBUF == 0
    n_rounds = n_chunks // NBUF

    mesh = plsc.VectorSubcoreMesh(core_axis_name="c", subcore_axis_name="s")

    @functools.partial(
        pl.kernel,
        out_type=jax.ShapeDtypeStruct((n, 128), jnp.float32),
        mesh=mesh,
        scratch_types=[
            pltpu.VMEM((n_chunks, CHUNK), jnp.int32),
            [pltpu.VMEM((CHUNK, d), jnp.float32) for _ in range(NBUF)],
            [pltpu.SemaphoreType.DMA for _ in range(NBUF)],
            [pltpu.SemaphoreType.DMA for _ in range(NBUF)],
        ],
        compiler_params=pltpu.CompilerParams(use_tc_tiling_on_sc=False),
    )
    def gather_kernel(idx_hbm, table_hbm, out_hbm, idx_all, rows, gsems,
                      osems):
        wid = lax.axis_index("s") * info.num_cores + lax.axis_index("c")
        base = wid * per_w

        pltpu.sync_copy(idx_hbm.at[wid], idx_all)

        def stage(k, slot):
            pltpu.make_async_copy(table_hbm.at[idx_all.at[k]], rows[slot],
                                  gsems[slot]).start()

        def wait_gather(slot):
            pltpu.make_async_copy(table_hbm.at[idx_all.at[0]], rows[slot],
                                  gsems[slot]).wait()

        def writeback(j, slot):
            pltpu.make_async_copy(rows[slot],
                                  out_hbm.at[pl.ds(base + j * CHUNK,
                                                    CHUNK), pl.ds(0, d)],
                                  osems[slot]).start()

        def wait_writeback(slot):
            pltpu.make_async_copy(rows[slot],
                                  out_hbm.at[pl.ds(base, CHUNK), pl.ds(0, d)],
                                  osems[slot]).wait()

        for s in range(NBUF - 1):
            stage(s, s)

        def round_body(r, carry):
            j0 = r * NBUF
            for s in range(NBUF):
                j = j0 + s
                prev_slot = (s - 1) % NBUF
                @pl.when(j >= 1)
                def _():
                    wait_writeback(prev_slot)

                @pl.when(j + NBUF - 1 < n_chunks)
                def _():
                    stage(j + NBUF - 1, prev_slot)

                wait_gather(s)
                writeback(j, s)
            return carry

        lax.fori_loop(0, n_rounds, round_body, 0)
        wait_writeback((n_chunks - 1) % NBUF)

    return gather_kernel


def kernel(words, word_embed_table):
    b, s = words.shape
    _, d = word_embed_table.shape
    n = b * s
    info = plsc.get_sparse_core_info()
    nw = info.num_cores * info.num_subcores
    per_w = n // nw
    idx3 = words.reshape(nw, per_w // CHUNK, CHUNK).astype(jnp.int32)
    n_words = word_embed_table.shape[0]
    out = _build(n, d)(idx3, word_embed_table)
    return out[:, :d].reshape(b, s, d)
